# baseline (speedup 1.0000x reference)
import functools

import jax
import jax.numpy as jnp
from jax import lax
from jax.experimental import pallas as pl
from jax.experimental.pallas import tpu as pltpu
from jax.experimental.pallas import tpu_sc as plsc

_NC = 2
_NS = 16
_NW = _NC * _NS


@functools.cache
def _make_prep(V, D):
    scale = jnp.float32(float(D) ** 0.5)
    n_full = V // 128
    tail = V - n_full * 128
    assert tail % 16 == 0 and D % 16 == 0
    mesh = plsc.VectorSubcoreMesh(core_axis_name="c", subcore_axis_name="s")

    @functools.partial(
        pl.kernel,
        out_type=jax.ShapeDtypeStruct((V // 4, 2 * D), jnp.int32),
        mesh=mesh,
        compiler_params=pltpu.CompilerParams(needs_layout_passes=False),
        scratch_types=[
            pltpu.VMEM((D, 128), jnp.float32),
            pltpu.VMEM((D, 128), jnp.float32),
            pltpu.VMEM((32, 128), jnp.int32),
            pltpu.VMEM((32, 128), jnp.int32),
            pltpu.SemaphoreType.DMA,
            pltpu.SemaphoreType.DMA,
            pltpu.SemaphoreType.DMA,
            pltpu.SemaphoreType.DMA,
        ],
    )
    def prep(tbl_hbm, tail_hbm, scr_hbm, tb0, tb1, sb0, sb1,
             rs0, rs1, ws0, ws1):
        wid = lax.axis_index("s") * _NC + lax.axis_index("c")
        tbuf = (tb0, tb1)
        sbuf = (sb0, sb1)
        rsem = (rs0, rs1)
        wsem = (ws0, ws1)
        it = lax.iota(jnp.int32, 16)
        row_quarter = it // 4
        col_p2 = (it % 4) * 32
        lane_skew = it % 16

        def fire_read(blk, b, width):
            pltpu.async_copy(
                tbl_hbm.at[:, pl.ds(blk * 128, width)],
                tbuf[b].at[:, pl.ds(0, width)],
                rsem[b],
            )

        def wait_read(blk, b, width):
            pltpu.make_async_copy(
                tbl_hbm.at[:, pl.ds(blk * 128, width)],
                tbuf[b].at[:, pl.ds(0, width)],
                rsem[b],
            ).wait()

        def transpose_block(b, width):
            n_vg = width // 16

            @plsc.parallel_loop(0, D // 2, unroll=8)
            def d_loop(d2):
                for vg in range(n_vg):
                    lo = tbuf[b][2 * d2, pl.ds(vg * 16, 16)] * scale
                    hi = tbuf[b][2 * d2 + 1, pl.ds(vg * 16, 16)] * scale
                    packed = plsc.pack(lo, hi, format=plsc.PackFormat.INTERLEAVED)
                    w = plsc.bitcast(packed, jnp.int32)
                    row_idx = row_quarter + (vg * 4)
                    col_idx = col_p2 + (d2 ^ lane_skew)
                    plsc.store_scatter(sbuf[b], [row_idx, col_idx], w)

        def fire_write(blk, b, width):
            pltpu.async_copy(
                sbuf[b].at[pl.ds(0, width // 4)],
                scr_hbm.at[pl.ds(blk * 32, width // 4)],
                wsem[b],
            )

        def wait_write(blk, b, width):
            pltpu.make_async_copy(
                sbuf[b].at[pl.ds(0, width // 4)],
                scr_hbm.at[pl.ds(blk * 32, width // 4)],
                wsem[b],
            ).wait()

        fire_read(wid, 0, 128)

        @pl.loop(wid, n_full, step=_NW)
        def blk_loop(blk):
            ordinal = (blk - wid) // _NW
            b0 = lax.rem(ordinal, 2)

            def do(b):
                nxt = blk + _NW

                @pl.when(nxt < n_full)
                def _():
                    fire_read(nxt, 1 - b, 128)

                wait_read(blk, b, 128)

                @pl.when(ordinal >= 2)
                def _():
                    wait_write(blk - 2 * _NW, b, 128)

                transpose_block(b, 128)
                fire_write(blk, b, 128)

            @pl.when(b0 == 0)
            def _():
                do(0)

            @pl.when(b0 == 1)
            def _():
                do(1)

        n_mine = (n_full - 1 - wid) // _NW + 1
        for back in (2, 1):
            ordinal = n_mine - back
            blk = wid + ordinal * _NW
            par = lax.rem(ordinal, 2)

            @pl.when(par == 0)
            def _():
                wait_write(blk, 0, 128)

            @pl.when(par == 1)
            def _():
                wait_write(blk, 1, 128)

        if tail:
            @pl.when(wid == n_full % _NW)
            def _():
                pltpu.sync_copy(tail_hbm, sbuf[0].at[pl.ds(0, tail // 4)])
                pltpu.sync_copy(
                    sbuf[0].at[pl.ds(0, tail // 4)],
                    scr_hbm.at[pl.ds(n_full * 32, tail // 4)],
                )

    return prep


@functools.cache
def _make_gather(N, T, V, D):
    assert D == 64
    assert T % 8 == 0 and N % 128 == 0
    n_ib = N // 128
    n_chunks = (T // 8) * n_ib
    assert n_chunks % _NW == 0
    per_w = n_chunks // _NW
    mesh = plsc.VectorSubcoreMesh(core_axis_name="c", subcore_axis_name="s")

    @functools.partial(
        pl.kernel,
        out_type=jax.ShapeDtypeStruct((T, D, N), jnp.float32),
        mesh=mesh,
        compiler_params=pltpu.CompilerParams(needs_layout_passes=False),
        scratch_types=[
            pltpu.VMEM((8, 128), jnp.int32),
            pltpu.VMEM((16, 64), jnp.int32),
            pltpu.VMEM((16, 64), jnp.int32),
            pltpu.VMEM((64, 128), jnp.int32),
            pltpu.VMEM((64, 128), jnp.int32),
            pltpu.VMEM((64, 128), jnp.int32),
            pltpu.VMEM((64, 128), jnp.int32),
            pltpu.VMEM((64, 128), jnp.float32),
            pltpu.VMEM((64, 128), jnp.float32),
            pltpu.SemaphoreType.DMA,
            pltpu.SemaphoreType.DMA,
            pltpu.SemaphoreType.DMA,
            pltpu.SemaphoreType.DMA,
            pltpu.SemaphoreType.DMA,
            pltpu.SemaphoreType.DMA,
            pltpu.SemaphoreType.DMA,
        ],
    )
    def gather(xt_hbm, scr_hbm, out_hbm, xbuf, kbuf, pbuf,
               pr0, pr1, pr2, pr3, ob0, ob1,
               xsem, gs0, gs1, gs2, gs3, ws0, ws1):
        wid = lax.axis_index("s") * _NC + lax.axis_index("c")
        pair = (pr0, pr1, pr2, pr3)
        gsem = (gs0, gs1, gs2, gs3)
        obuf = (ob0, ob1)
        wsem = (ws0, ws1)
        it = lax.iota(jnp.int32, 16)
        rows = [it + (ig * 16) for ig in range(4)]

        def fire_gather(th, r):
            pltpu.async_copy(scr_hbm.at[kbuf.at[th]], pair[r], gsem[r])

        def wait_gather(th, r):
            pltpu.make_async_copy(
                scr_hbm.at[kbuf.at[th]], pair[r], gsem[r]
            ).wait()

        def fire_wb(t_glob, ib, b):
            pltpu.async_copy(
                obuf[b],
                out_hbm.at[t_glob, :, pl.ds(ib * 128, 128)],
                wsem[b],
            )

        def wait_wb(t_glob, ib, b):
            pltpu.make_async_copy(
                obuf[b],
                out_hbm.at[t_glob, :, pl.ds(ib * 128, 128)],
                wsem[b],
            ).wait()

        def transpose_half(th, r, b):
            q = th % 2
            for ig in range(4):
                pvec = pbuf[th, pl.ds(ig * 16, 16)]
                kvec = kbuf[th, pl.ds(ig * 16, 16)]
                skew = ((kvec % 4) * 4) ^ pvec
                base = pvec * 32
                obase = q * 64 + ig * 16

                @plsc.parallel_loop(0, D // 2, unroll=8)
                def d_loop(d2):
                    col = base + (d2 ^ skew)
                    w = plsc.load_gather(pair[r], [rows[ig], col])
                    bf = plsc.bitcast(w, jnp.bfloat16)
                    lo, hi = plsc.unpack(
                        bf,
                        format=plsc.PackFormat.INTERLEAVED,
                        preferred_element_type=jnp.float32,
                    )
                    obuf[b][2 * d2, pl.ds(obase, 16)] = lo
                    obuf[b][2 * d2 + 1, pl.ds(obase, 16)] = hi

        @pl.loop(0, per_w)
        def chunk_loop(ci):
            chunk = wid + ci * _NW
            tg = chunk // n_ib
            ib = lax.rem(chunk, n_ib)

            pltpu.sync_copy(
                xt_hbm.at[pl.ds(tg * 8, 8), pl.ds(ib * 128, 128)], xbuf
            )
            for t in range(8):
                for g in range(8):
                    v = xbuf[t, pl.ds(g * 16, 16)]
                    th = t * 2 + g // 4
                    colk = (g % 4) * 16
                    kbuf[th, pl.ds(colk, 16)] = v // 4
                    pbuf[th, pl.ds(colk, 16)] = lax.rem(v, 4)

            fire_gather(0, 0)
            fire_gather(1, 1)
            for t in range(8):
                for q in range(2):
                    th = t * 2 + q
                    r = th % 4
                    b = t % 2
                    if th in (0, 2) :
                        @pl.when(ci > 0)
                        def _():
                            prev = wid + (ci - 1) * _NW
                            wait_wb(
                                (prev // n_ib) * 8 + 6 + b,
                                lax.rem(prev, n_ib),
                                b,
                            )
                    if th >= 4 and q == 0:
                        wait_wb(tg * 8 + t - 2, ib, b)
                    if th + 2 < 16:
                        fire_gather(th + 2, (th + 2) % 4)
                    wait_gather(th, r)
                    transpose_half(th, r, b)
                    if q == 1:
                        fire_wb(tg * 8 + t, ib, b)

        last = wid + (per_w - 1) * _NW
        wait_wb((last // n_ib) * 8 + 6, lax.rem(last, n_ib), 0)
        wait_wb((last // n_ib) * 8 + 7, lax.rem(last, n_ib), 1)

    return gather


def kernel(x, table):
    N, T = x.shape
    V, D = table.shape
    n_full = V // 128
    tail = V - n_full * 128
    scale = jnp.float32(float(D) ** 0.5)
    tb = (table[V - tail:, :] * scale).astype(jnp.bfloat16)
    tw = jax.lax.bitcast_convert_type(
        tb.reshape(tail, D // 2, 2), jnp.int32
    ).reshape(tail // 4, 4, D // 2)
    kl = jnp.arange(tail // 4, dtype=jnp.int32)[:, None, None]
    pp = jnp.arange(4, dtype=jnp.int32)[None, :, None]
    dd = jnp.arange(D // 2, dtype=jnp.int32)[None, None, :]
    cols = pp * (D // 2) + (dd ^ (((kl % 4) * 4) ^ pp))
    tail_scr = (
        jnp.zeros((tail // 4, 2 * D), jnp.int32)
        .at[kl, cols]
        .set(tw)
    )
    scr = _make_prep(V, D)(table.T, tail_scr)
    out_t = _make_gather(N, T, V, D)(x.T.astype(jnp.int32), scr)
    return out_t.transpose(2, 0, 1)


# speedup vs baseline: 1.0692x; 1.0692x over previous
import functools

import jax
import jax.numpy as jnp
from jax import lax
from jax.experimental import pallas as pl
from jax.experimental.pallas import tpu as pltpu
from jax.experimental.pallas import tpu_sc as plsc

_NC = 2
_NS = 16
_NW = _NC * _NS


@functools.cache
def _make_prep(V, D):
    scale = jnp.float32(float(D) ** 0.5)
    n_full = V // 128
    tail = V - n_full * 128
    assert tail % 16 == 0 and D % 16 == 0
    mesh = plsc.VectorSubcoreMesh(core_axis_name="c", subcore_axis_name="s")

    @functools.partial(
        pl.kernel,
        out_type=jax.ShapeDtypeStruct((V // 2, 2 * D), jnp.float32),
        mesh=mesh,
        compiler_params=pltpu.CompilerParams(needs_layout_passes=False),
        scratch_types=[
            pltpu.VMEM((D, 128), jnp.float32),
            pltpu.VMEM((D, 128), jnp.float32),
            pltpu.VMEM((64, 128), jnp.float32),
            pltpu.VMEM((64, 128), jnp.float32),
            pltpu.SemaphoreType.DMA,
            pltpu.SemaphoreType.DMA,
            pltpu.SemaphoreType.DMA,
            pltpu.SemaphoreType.DMA,
        ],
    )
    def prep(tbl_hbm, tail_hbm, scr_hbm, tb0, tb1, sb0, sb1,
             rs0, rs1, ws0, ws1):
        wid = lax.axis_index("s") * _NC + lax.axis_index("c")
        tbuf = (tb0, tb1)
        sbuf = (sb0, sb1)
        rsem = (rs0, rs1)
        wsem = (ws0, ws1)
        it = lax.iota(jnp.int32, 16)
        row_half = it // 2
        col_par = (it % 2) * 64
        par8 = (it % 2) * 8
        skews = [
            (((vg * 8 + row_half) % 16) ^ par8) for vg in range(8)
        ]

        def fire_read(blk, b, width):
            pltpu.async_copy(
                tbl_hbm.at[:, pl.ds(blk * 128, width)],
                tbuf[b].at[:, pl.ds(0, width)],
                rsem[b],
            )

        def wait_read(blk, b, width):
            pltpu.make_async_copy(
                tbl_hbm.at[:, pl.ds(blk * 128, width)],
                tbuf[b].at[:, pl.ds(0, width)],
                rsem[b],
            ).wait()

        def transpose_block(b, width):
            n_vg = width // 16

            @plsc.parallel_loop(0, D, unroll=8)
            def d_loop(d):
                for vg in range(n_vg):
                    vals = tbuf[b][d, pl.ds(vg * 16, 16)] * scale
                    row_idx = row_half + (vg * 8)
                    col_idx = (col_par + d) ^ skews[vg]
                    plsc.store_scatter(sbuf[b], [row_idx, col_idx], vals)

        def fire_write(blk, b, width):
            pltpu.async_copy(
                sbuf[b].at[pl.ds(0, width // 2)],
                scr_hbm.at[pl.ds(blk * 64, width // 2)],
                wsem[b],
            )

        def wait_write(blk, b, width):
            pltpu.make_async_copy(
                sbuf[b].at[pl.ds(0, width // 2)],
                scr_hbm.at[pl.ds(blk * 64, width // 2)],
                wsem[b],
            ).wait()

        fire_read(wid, 0, 128)

        @pl.loop(wid, n_full, step=_NW)
        def blk_loop(blk):
            ordinal = (blk - wid) // _NW
            b0 = lax.rem(ordinal, 2)

            def do(b):
                nxt = blk + _NW

                @pl.when(nxt < n_full)
                def _():
                    fire_read(nxt, 1 - b, 128)

                wait_read(blk, b, 128)

                @pl.when(ordinal >= 2)
                def _():
                    wait_write(blk - 2 * _NW, b, 128)

                transpose_block(b, 128)
                fire_write(blk, b, 128)

            @pl.when(b0 == 0)
            def _():
                do(0)

            @pl.when(b0 == 1)
            def _():
                do(1)

        n_mine = (n_full - 1 - wid) // _NW + 1
        for back in (2, 1):
            ordinal = n_mine - back
            blk = wid + ordinal * _NW
            par = lax.rem(ordinal, 2)

            @pl.when(par == 0)
            def _():
                wait_write(blk, 0, 128)

            @pl.when(par == 1)
            def _():
                wait_write(blk, 1, 128)

        if tail:
            @pl.when(wid == n_full % _NW)
            def _():
                pltpu.sync_copy(tail_hbm, sbuf[0].at[pl.ds(0, tail // 2)])
                pltpu.sync_copy(
                    sbuf[0].at[pl.ds(0, tail // 2)],
                    scr_hbm.at[pl.ds(n_full * 64, tail // 2)],
                )

    return prep


@functools.cache
def _make_gather(N, T, V, D):
    assert D == 64
    assert T % 8 == 0 and N % 128 == 0
    n_ib = N // 128
    n_chunks = (T // 8) * n_ib
    assert n_chunks % _NW == 0
    per_w = n_chunks // _NW
    mesh = plsc.VectorSubcoreMesh(core_axis_name="c", subcore_axis_name="s")

    @functools.partial(
        pl.kernel,
        out_type=jax.ShapeDtypeStruct((T, D, N), jnp.float32),
        mesh=mesh,
        compiler_params=pltpu.CompilerParams(needs_layout_passes=False),
        scratch_types=[
            pltpu.VMEM((8, 128), jnp.int32),
            pltpu.VMEM((16, 64), jnp.int32),
            pltpu.VMEM((16, 64), jnp.int32),
            pltpu.VMEM((64, 128), jnp.float32),
            pltpu.VMEM((64, 128), jnp.float32),
            pltpu.VMEM((64, 128), jnp.float32),
            pltpu.VMEM((64, 128), jnp.float32),
            pltpu.VMEM((64, 128), jnp.float32),
            pltpu.VMEM((64, 128), jnp.float32),
            pltpu.SemaphoreType.DMA,
            pltpu.SemaphoreType.DMA,
            pltpu.SemaphoreType.DMA,
            pltpu.SemaphoreType.DMA,
            pltpu.SemaphoreType.DMA,
            pltpu.SemaphoreType.DMA,
            pltpu.SemaphoreType.DMA,
        ],
    )
    def gather(xt_hbm, scr_hbm, out_hbm, xbuf, kbuf, pbuf,
               pr0, pr1, pr2, pr3, ob0, ob1,
               xsem, gs0, gs1, gs2, gs3, ws0, ws1):
        wid = lax.axis_index("s") * _NC + lax.axis_index("c")
        pair = (pr0, pr1, pr2, pr3)
        gsem = (gs0, gs1, gs2, gs3)
        obuf = (ob0, ob1)
        wsem = (ws0, ws1)
        it = lax.iota(jnp.int32, 16)
        rows = [it + (ig * 16) for ig in range(4)]

        def fire_gather(th, r):
            pltpu.async_copy(scr_hbm.at[kbuf.at[th]], pair[r], gsem[r])

        def wait_gather(th, r):
            pltpu.make_async_copy(
                scr_hbm.at[kbuf.at[th]], pair[r], gsem[r]
            ).wait()

        def fire_wb(t_glob, ib, b):
            pltpu.async_copy(
                obuf[b],
                out_hbm.at[t_glob, :, pl.ds(ib * 128, 128)],
                wsem[b],
            )

        def wait_wb(t_glob, ib, b):
            pltpu.make_async_copy(
                obuf[b],
                out_hbm.at[t_glob, :, pl.ds(ib * 128, 128)],
                wsem[b],
            ).wait()

        def transpose_half(th, r, b):
            q = th % 2
            for ig in range(4):
                pvec = pbuf[th, pl.ds(ig * 16, 16)]
                kvec = kbuf[th, pl.ds(ig * 16, 16)]
                skew = (kvec % 16) ^ (pvec * 8)
                base = pvec * 64
                obase = q * 64 + ig * 16

                @plsc.parallel_loop(0, D, unroll=8)
                def d_loop(d):
                    col = (base + d) ^ skew
                    vals = plsc.load_gather(pair[r], [rows[ig], col])
                    obuf[b][d, pl.ds(obase, 16)] = vals

        @pl.loop(0, per_w)
        def chunk_loop(ci):
            chunk = wid + ci * _NW
            tg = chunk // n_ib
            ib = lax.rem(chunk, n_ib)

            pltpu.sync_copy(
                xt_hbm.at[pl.ds(tg * 8, 8), pl.ds(ib * 128, 128)], xbuf
            )
            for t in range(8):
                for g in range(8):
                    v = xbuf[t, pl.ds(g * 16, 16)]
                    th = t * 2 + g // 4
                    colk = (g % 4) * 16
                    kbuf[th, pl.ds(colk, 16)] = v // 2
                    pbuf[th, pl.ds(colk, 16)] = lax.rem(v, 2)

            fire_gather(0, 0)
            fire_gather(1, 1)
            fire_gather(2, 2)
            for t in range(8):
                for q in range(2):
                    th = t * 2 + q
                    r = th % 4
                    b = t % 2
                    if th in (0, 2) :
                        @pl.when(ci > 0)
                        def _():
                            prev = wid + (ci - 1) * _NW
                            wait_wb(
                                (prev // n_ib) * 8 + 6 + b,
                                lax.rem(prev, n_ib),
                                b,
                            )
                    if th >= 4 and q == 0:
                        wait_wb(tg * 8 + t - 2, ib, b)
                    if th + 3 < 16:
                        fire_gather(th + 3, (th + 3) % 4)
                    wait_gather(th, r)
                    transpose_half(th, r, b)
                    if q == 1:
                        fire_wb(tg * 8 + t, ib, b)

        last = wid + (per_w - 1) * _NW
        wait_wb((last // n_ib) * 8 + 6, lax.rem(last, n_ib), 0)
        wait_wb((last // n_ib) * 8 + 7, lax.rem(last, n_ib), 1)

    return gather


def kernel(x, table):
    N, T = x.shape
    V, D = table.shape
    n_full = V // 128
    tail = V - n_full * 128
    scale = jnp.float32(float(D) ** 0.5)
    tv = (table[V - tail:, :] * scale).reshape(tail // 2, 2, D)
    kl = jnp.arange(tail // 2, dtype=jnp.int32)[:, None, None]
    pp = jnp.arange(2, dtype=jnp.int32)[None, :, None]
    dd = jnp.arange(D, dtype=jnp.int32)[None, None, :]
    k_glob = (V // 128) * 64 + kl
    cols = (pp * D + dd) ^ ((k_glob % 16) ^ (pp * 8))
    tail_scr = (
        jnp.zeros((tail // 2, 2 * D), jnp.float32)
        .at[kl, cols]
        .set(tv)
    )
    scr = _make_prep(V, D)(table.T, tail_scr)
    out_t = _make_gather(N, T, V, D)(x.T.astype(jnp.int32), scr)
    return out_t.transpose(2, 0, 1)


# speedup vs baseline: 1.0729x; 1.0035x over previous
import functools

import jax
import jax.numpy as jnp
from jax import lax
from jax.experimental import pallas as pl
from jax.experimental.pallas import tpu as pltpu
from jax.experimental.pallas import tpu_sc as plsc

_NC = 2
_NS = 16
_NW = _NC * _NS


@functools.cache
def _make_prep(V, D):
    scale = jnp.float32(float(D) ** 0.5)
    n_full = V // 128
    tail = V - n_full * 128
    assert tail % 16 == 0 and D % 16 == 0
    mesh = plsc.VectorSubcoreMesh(core_axis_name="c", subcore_axis_name="s")

    @functools.partial(
        pl.kernel,
        out_type=jax.ShapeDtypeStruct((V // 2, 2 * D), jnp.float32),
        mesh=mesh,
        compiler_params=pltpu.CompilerParams(needs_layout_passes=False),
        scratch_types=[
            pltpu.VMEM((D, 128), jnp.float32),
            pltpu.VMEM((D, 128), jnp.float32),
            pltpu.VMEM((64, 128), jnp.float32),
            pltpu.VMEM((64, 128), jnp.float32),
            pltpu.SemaphoreType.DMA,
            pltpu.SemaphoreType.DMA,
            pltpu.SemaphoreType.DMA,
            pltpu.SemaphoreType.DMA,
        ],
    )
    def prep(tbl_hbm, tail_hbm, scr_hbm, tb0, tb1, sb0, sb1,
             rs0, rs1, ws0, ws1):
        wid = lax.axis_index("s") * _NC + lax.axis_index("c")
        tbuf = (tb0, tb1)
        sbuf = (sb0, sb1)
        rsem = (rs0, rs1)
        wsem = (ws0, ws1)
        it = lax.iota(jnp.int32, 16)
        row_half = it // 2
        col_par = (it % 2) * 64
        par8 = (it % 2) * 8
        skews = [
            (((vg * 8 + row_half) % 16) ^ par8) for vg in range(8)
        ]

        def fire_read(blk, b, width):
            pltpu.async_copy(
                tbl_hbm.at[:, pl.ds(blk * 128, width)],
                tbuf[b].at[:, pl.ds(0, width)],
                rsem[b],
            )

        def wait_read(blk, b, width):
            pltpu.make_async_copy(
                tbl_hbm.at[:, pl.ds(blk * 128, width)],
                tbuf[b].at[:, pl.ds(0, width)],
                rsem[b],
            ).wait()

        def transpose_block(b, width):
            n_vg = width // 16

            @plsc.parallel_loop(0, D, unroll=8)
            def d_loop(d):
                for vg in range(n_vg):
                    vals = tbuf[b][d, pl.ds(vg * 16, 16)] * scale
                    row_idx = row_half + (vg * 8)
                    col_idx = (col_par + d) ^ skews[vg]
                    plsc.store_scatter(sbuf[b], [row_idx, col_idx], vals)

        def fire_write(blk, b, width):
            pltpu.async_copy(
                sbuf[b].at[pl.ds(0, width // 2)],
                scr_hbm.at[pl.ds(blk * 64, width // 2)],
                wsem[b],
            )

        def wait_write(blk, b, width):
            pltpu.make_async_copy(
                sbuf[b].at[pl.ds(0, width // 2)],
                scr_hbm.at[pl.ds(blk * 64, width // 2)],
                wsem[b],
            ).wait()

        fire_read(wid, 0, 128)

        @pl.loop(wid, n_full, step=_NW)
        def blk_loop(blk):
            ordinal = (blk - wid) // _NW
            b0 = lax.rem(ordinal, 2)

            def do(b):
                nxt = blk + _NW

                @pl.when(nxt < n_full)
                def _():
                    fire_read(nxt, 1 - b, 128)

                wait_read(blk, b, 128)

                @pl.when(ordinal >= 2)
                def _():
                    wait_write(blk - 2 * _NW, b, 128)

                transpose_block(b, 128)
                fire_write(blk, b, 128)

            @pl.when(b0 == 0)
            def _():
                do(0)

            @pl.when(b0 == 1)
            def _():
                do(1)

        n_mine = (n_full - 1 - wid) // _NW + 1
        for back in (2, 1):
            ordinal = n_mine - back
            blk = wid + ordinal * _NW
            par = lax.rem(ordinal, 2)

            @pl.when(par == 0)
            def _():
                wait_write(blk, 0, 128)

            @pl.when(par == 1)
            def _():
                wait_write(blk, 1, 128)

        if tail:
            @pl.when(wid == n_full % _NW)
            def _():
                pltpu.sync_copy(tail_hbm, sbuf[0].at[pl.ds(0, tail // 2)])
                pltpu.sync_copy(
                    sbuf[0].at[pl.ds(0, tail // 2)],
                    scr_hbm.at[pl.ds(n_full * 64, tail // 2)],
                )

    return prep


@functools.cache
def _make_gather(N, T, V, D):
    assert D == 64
    assert T % 8 == 0 and N % 128 == 0
    n_ib = N // 128
    n_chunks = (T // 8) * n_ib
    assert n_chunks % _NW == 0
    per_w = n_chunks // _NW
    mesh = plsc.VectorSubcoreMesh(core_axis_name="c", subcore_axis_name="s")

    @functools.partial(
        pl.kernel,
        out_type=jax.ShapeDtypeStruct((T, D, N), jnp.float32),
        mesh=mesh,
        compiler_params=pltpu.CompilerParams(needs_layout_passes=False),
        scratch_types=[
            pltpu.VMEM((8, 128), jnp.int32),
            pltpu.VMEM((16, 64), jnp.int32),
            pltpu.VMEM((16, 64), jnp.int32),
            pltpu.VMEM((64, 128), jnp.float32),
            pltpu.VMEM((64, 128), jnp.float32),
            pltpu.VMEM((64, 128), jnp.float32),
            pltpu.VMEM((64, 128), jnp.float32),
            pltpu.VMEM((64, 128), jnp.float32),
            pltpu.VMEM((64, 128), jnp.float32),
            pltpu.VMEM((64, 128), jnp.float32),
            pltpu.VMEM((64, 128), jnp.float32),
            pltpu.VMEM((64, 128), jnp.float32),
            pltpu.VMEM((64, 128), jnp.float32),
            pltpu.SemaphoreType.DMA,
            pltpu.SemaphoreType.DMA,
            pltpu.SemaphoreType.DMA,
            pltpu.SemaphoreType.DMA,
            pltpu.SemaphoreType.DMA,
            pltpu.SemaphoreType.DMA,
            pltpu.SemaphoreType.DMA,
            pltpu.SemaphoreType.DMA,
            pltpu.SemaphoreType.DMA,
            pltpu.SemaphoreType.DMA,
            pltpu.SemaphoreType.DMA,
        ],
    )
    def gather(xt_hbm, scr_hbm, out_hbm, xbuf, kbuf, pbuf,
               pr0, pr1, pr2, pr3, pr4, pr5, pr6, pr7, ob0, ob1,
               xsem, gs0, gs1, gs2, gs3, gs4, gs5, gs6, gs7, ws0, ws1):
        wid = lax.axis_index("s") * _NC + lax.axis_index("c")
        pair = (pr0, pr1, pr2, pr3, pr4, pr5, pr6, pr7)
        gsem = (gs0, gs1, gs2, gs3, gs4, gs5, gs6, gs7)
        obuf = (ob0, ob1)
        wsem = (ws0, ws1)
        it = lax.iota(jnp.int32, 16)
        rows = [it + (ig * 16) for ig in range(4)]

        def fire_gather(th, r):
            pltpu.async_copy(scr_hbm.at[kbuf.at[th]], pair[r], gsem[r])

        def wait_gather(th, r):
            pltpu.make_async_copy(
                scr_hbm.at[kbuf.at[th]], pair[r], gsem[r]
            ).wait()

        def fire_wb(t_glob, ib, b):
            pltpu.async_copy(
                obuf[b],
                out_hbm.at[t_glob, :, pl.ds(ib * 128, 128)],
                wsem[b],
            )

        def wait_wb(t_glob, ib, b):
            pltpu.make_async_copy(
                obuf[b],
                out_hbm.at[t_glob, :, pl.ds(ib * 128, 128)],
                wsem[b],
            ).wait()

        def transpose_half(th, r, b):
            q = th % 2
            for ig in range(4):
                pvec = pbuf[th, pl.ds(ig * 16, 16)]
                kvec = kbuf[th, pl.ds(ig * 16, 16)]
                skew = (kvec % 16) ^ (pvec * 8)
                base = pvec * 64
                obase = q * 64 + ig * 16

                @plsc.parallel_loop(0, D, unroll=8)
                def d_loop(d):
                    col = (base + d) ^ skew
                    vals = plsc.load_gather(pair[r], [rows[ig], col])
                    obuf[b][d, pl.ds(obase, 16)] = vals

        @pl.loop(0, per_w)
        def chunk_loop(ci):
            chunk = wid + ci * _NW
            tg = chunk // n_ib
            ib = lax.rem(chunk, n_ib)

            pltpu.sync_copy(
                xt_hbm.at[pl.ds(tg * 8, 8), pl.ds(ib * 128, 128)], xbuf
            )
            for t in range(8):
                for g in range(8):
                    v = xbuf[t, pl.ds(g * 16, 16)]
                    th = t * 2 + g // 4
                    colk = (g % 4) * 16
                    kbuf[th, pl.ds(colk, 16)] = v // 2
                    pbuf[th, pl.ds(colk, 16)] = lax.rem(v, 2)

            for th0 in range(6):
                fire_gather(th0, th0)
            for t in range(8):
                for q in range(2):
                    th = t * 2 + q
                    r = th % 8
                    b = t % 2
                    if th in (0, 2) :
                        @pl.when(ci > 0)
                        def _():
                            prev = wid + (ci - 1) * _NW
                            wait_wb(
                                (prev // n_ib) * 8 + 6 + b,
                                lax.rem(prev, n_ib),
                                b,
                            )
                    if th >= 4 and q == 0:
                        wait_wb(tg * 8 + t - 2, ib, b)
                    if th + 6 < 16:
                        fire_gather(th + 6, (th + 6) % 8)
                    wait_gather(th, r)
                    transpose_half(th, r, b)
                    if q == 1:
                        fire_wb(tg * 8 + t, ib, b)

        last = wid + (per_w - 1) * _NW
        wait_wb((last // n_ib) * 8 + 6, lax.rem(last, n_ib), 0)
        wait_wb((last // n_ib) * 8 + 7, lax.rem(last, n_ib), 1)

    return gather


def kernel(x, table):
    N, T = x.shape
    V, D = table.shape
    n_full = V // 128
    tail = V - n_full * 128
    scale = jnp.float32(float(D) ** 0.5)
    tv = (table[V - tail:, :] * scale).reshape(tail // 2, 2, D)
    kl = jnp.arange(tail // 2, dtype=jnp.int32)[:, None, None]
    pp = jnp.arange(2, dtype=jnp.int32)[None, :, None]
    dd = jnp.arange(D, dtype=jnp.int32)[None, None, :]
    k_glob = (V // 128) * 64 + kl
    cols = (pp * D + dd) ^ ((k_glob % 16) ^ (pp * 8))
    tail_scr = (
        jnp.zeros((tail // 2, 2 * D), jnp.float32)
        .at[kl, cols]
        .set(tv)
    )
    scr = _make_prep(V, D)(table.T, tail_scr)
    out_t = _make_gather(N, T, V, D)(x.T.astype(jnp.int32), scr)
    return out_t.transpose(2, 0, 1)
